# baseline (speedup 1.0000x reference)
import functools

import jax
import jax.numpy as jnp
from jax import lax
from jax.experimental import pallas as pl
from jax.experimental.pallas import tpu as pltpu
from jax.experimental.pallas import tpu_sc as plsc

G = 1024
CHUNK = 64
NHP = 2
NW = 32


def _lane_bcast(v16, i):
    return lax.gather(
        v16,
        jnp.full((16, 1), i, jnp.int32),
        lax.GatherDimensionNumbers(
            offset_dims=(),
            collapsed_slice_dims=(0,),
            start_index_map=(0,),
        ),
        slice_sizes=(1,),
        mode=lax.GatherScatterMode.PROMISE_IN_BOUNDS,
    )


def _sc_body(B, S, H, feats_hbm, gids_hbm, vals_hbm, out_hbm,
             buf, ibuf, vbuf, gscan, acc):
    c = lax.axis_index("c")
    s = lax.axis_index("s")
    w = c * 16 + s

    wpb = NW // B
    gpw = G // wpb
    HP = H // NHP
    hv = HP // 16

    bw = w // wpb
    gr0 = (w % wpb) * gpw
    tok_base = bw * S

    pltpu.sync_copy(gids_hbm.at[pl.ds(tok_base, S)], gscan)
    lo_t = jnp.full((16,), gr0, jnp.int32)
    hi_t = jnp.full((16,), gr0 + gpw, jnp.int32)

    one = jnp.ones((16,), jnp.int32)
    zero = jnp.zeros((16,), jnp.int32)

    def count_step(i, carry):
        lo, hi = carry
        v = gscan[pl.ds(i * 16, 16)]
        nlo = jnp.where(v < lo_t, one, zero)
        nhi = jnp.where(v < hi_t, one, zero)
        return lo + nlo, hi + nhi

    lo_cnt, hi_cnt = lax.fori_loop(0, S // 16, count_step, (zero, zero))
    s_lo = jnp.int32(0)
    s_hi = jnp.int32(0)
    for j in range(16):
        s_lo = s_lo + lo_cnt[j]
        s_hi = s_hi + hi_cnt[j]

    ch_lo = s_lo // CHUNK
    ch_hi = (s_hi + CHUNK - 1) // CHUNK
    zvec = jnp.zeros((16,), jnp.float32)
    lo16 = jnp.full((16,), s_lo, jnp.int32)
    hi16 = jnp.full((16,), s_hi, jnp.int32)
    gr0_16 = jnp.full((16,), gr0, jnp.int32)

    for hp in range(NHP):
        col0 = hp * HP

        def zero_row(t, _):
            for k in range(hv):
                acc[t, pl.ds(k * 16, 16)] = zvec
            return 0

        lax.fori_loop(0, gpw, zero_row, 0)

        def chunk_step(ch, _):
            row0 = tok_base + ch * CHUNK
            pltpu.sync_copy(
                feats_hbm.at[pl.ds(row0, CHUNK), pl.ds(col0, HP)], buf)
            pltpu.sync_copy(gids_hbm.at[pl.ds(row0, CHUNK)], ibuf)
            pltpu.sync_copy(vals_hbm.at[pl.ds(row0, CHUNK)], vbuf)

            def block_step(jj, _):
                g16 = ibuf[pl.ds(jj * 16, 16)]
                v16 = vbuf[pl.ds(jj * 16, 16)]
                tok16 = (ch * CHUNK + jj * 16
                         + lax.iota(jnp.int32, 16))
                m = (tok16 >= lo16) & (tok16 < hi16)
                val16 = jnp.where(m, v16, 0.0)
                r16 = jnp.clip(g16 - gr0_16, 0, gpw - 1)

                for i in range(16):
                    valv = jnp.full((16,), val16[i], jnp.float32)
                    r = r16[i]
                    t = jj * 16 + i
                    for k in range(hv):
                        plsc.addupdate(
                            acc.at[r, pl.ds(k * 16, 16)],
                            buf[t, pl.ds(k * 16, 16)] * valv)
                return 0

            lax.fori_loop(0, CHUNK // 16, block_step, 0)
            return 0

        lax.fori_loop(ch_lo, ch_hi, chunk_step, 0)

        pltpu.sync_copy(
            acc, out_hbm.at[pl.ds(bw * G + gr0, gpw), pl.ds(col0, HP)])


def kernel(feats, indices, values):
    B, S, H = feats.shape
    feats_r = feats.reshape(B * S, H)
    gids = indices[1].astype(jnp.int32)
    vals = values.astype(jnp.float32)

    mesh = plsc.VectorSubcoreMesh(core_axis_name="c", subcore_axis_name="s")
    run = pl.kernel(
        functools.partial(_sc_body, B, S, H),
        out_type=jax.ShapeDtypeStruct((B * G, H), jnp.float32),
        mesh=mesh,
        compiler_params=pltpu.CompilerParams(use_tc_tiling_on_sc=False, needs_layout_passes=False),
        scratch_types=[
            pltpu.VMEM((CHUNK, H // NHP), jnp.float32),
            pltpu.VMEM((CHUNK,), jnp.int32),
            pltpu.VMEM((CHUNK,), jnp.float32),
            pltpu.VMEM((S,), jnp.int32),
            pltpu.VMEM((G // (NW // B), H // NHP), jnp.float32),
        ],
    )
    out = run(feats_r, gids, vals)
    return out.reshape(B, G, H)


# speedup vs baseline: 1.0278x; 1.0278x over previous
import functools

import jax
import jax.numpy as jnp
from jax import lax
from jax.experimental import pallas as pl
from jax.experimental.pallas import tpu as pltpu
from jax.experimental.pallas import tpu_sc as plsc

G = 1024
CHUNK = 64
NHP = 2
NW = 32


def _sc_body(B, S, H, feats_hbm, gids_hbm, vals_hbm, out_hbm,
             buf0, buf1, gscan, vscan, acc, sem0, sem1):
    c = lax.axis_index("c")
    s = lax.axis_index("s")
    w = c * 16 + s

    wpb = NW // B
    gpw = G // wpb
    HP = H // NHP
    hv = HP // 16
    nch = S // CHUNK

    bw = w // wpb
    gr0 = (w % wpb) * gpw
    tok_base = bw * S

    pltpu.sync_copy(gids_hbm.at[pl.ds(tok_base, S)], gscan)
    pltpu.sync_copy(vals_hbm.at[pl.ds(tok_base, S)], vscan)
    lo_t = jnp.full((16,), gr0, jnp.int32)
    hi_t = jnp.full((16,), gr0 + gpw, jnp.int32)
    one = jnp.ones((16,), jnp.int32)
    zero = jnp.zeros((16,), jnp.int32)

    def count_step(i, carry):
        lo, hi = carry
        v = gscan[pl.ds(i * 16, 16)]
        return (lo + jnp.where(v < lo_t, one, zero),
                hi + jnp.where(v < hi_t, one, zero))

    lo_cnt, hi_cnt = lax.fori_loop(0, S // 16, count_step, (zero, zero))
    s_lo = jnp.int32(0)
    s_hi = jnp.int32(0)
    for j in range(16):
        s_lo = s_lo + lo_cnt[j]
        s_hi = s_hi + hi_cnt[j]

    p_lo = s_lo // (2 * CHUNK)
    p_hi = (s_hi + 2 * CHUNK - 1) // (2 * CHUNK)
    last_ch = nch - 1
    zvec = jnp.zeros((16,), jnp.float32)
    lo16 = jnp.full((16,), s_lo, jnp.int32)
    hi16 = jnp.full((16,), s_hi, jnp.int32)
    gr0_16 = jnp.full((16,), gr0, jnp.int32)

    for hp in range(NHP):
        col0 = hp * HP

        def zero_row(t, _):
            for k in range(hv):
                acc[t, pl.ds(k * 16, 16)] = zvec
            return 0

        lax.fori_loop(0, gpw, zero_row, 0)

        def issue(ch, buf, sem):
            chc = jnp.minimum(ch, last_ch)
            pltpu.async_copy(
                feats_hbm.at[pl.ds(tok_base + chc * CHUNK, CHUNK),
                             pl.ds(col0, HP)],
                buf, sem)

        def wait(buf, sem):
            pltpu.make_async_copy(
                feats_hbm.at[pl.ds(0, CHUNK), pl.ds(col0, HP)],
                buf, sem).wait()

        def compute(ch, buf):
            def block_step(jj, _):
                off = ch * CHUNK + jj * 16
                g16 = gscan[pl.ds(off, 16)]
                v16 = vscan[pl.ds(off, 16)]
                tok16 = off + lax.iota(jnp.int32, 16)
                m = (tok16 >= lo16) & (tok16 < hi16)
                val16 = jnp.where(m, v16, 0.0)
                r16 = jnp.clip(g16 - gr0_16, 0, gpw - 1)
                for i in range(16):
                    valv = jnp.full((16,), val16[i], jnp.float32)
                    r = r16[i]
                    t = jj * 16 + i
                    for k in range(hv):
                        plsc.addupdate(
                            acc.at[r, pl.ds(k * 16, 16)],
                            buf[t, pl.ds(k * 16, 16)] * valv)
                return 0

            lax.fori_loop(0, CHUNK // 16, block_step, 0)

        issue(p_lo * 2, buf0, sem0)
        issue(p_lo * 2 + 1, buf1, sem1)

        def pair_step(p, _):
            ch0 = p * 2
            wait(buf0, sem0)
            compute(ch0, buf0)
            issue(ch0 + 2, buf0, sem0)
            wait(buf1, sem1)
            compute(ch0 + 1, buf1)
            issue(ch0 + 3, buf1, sem1)
            return 0

        lax.fori_loop(p_lo, p_hi, pair_step, 0)
        wait(buf0, sem0)
        wait(buf1, sem1)

        pltpu.sync_copy(
            acc, out_hbm.at[pl.ds(bw * G + gr0, gpw), pl.ds(col0, HP)])


def kernel(feats, indices, values):
    B, S, H = feats.shape
    feats_r = feats.reshape(B * S, H)
    gids = indices[1].astype(jnp.int32)
    vals = values.astype(jnp.float32)

    mesh = plsc.VectorSubcoreMesh(core_axis_name="c", subcore_axis_name="s")
    run = pl.kernel(
        functools.partial(_sc_body, B, S, H),
        out_type=jax.ShapeDtypeStruct((B * G, H), jnp.float32),
        mesh=mesh,
        compiler_params=pltpu.CompilerParams(
            use_tc_tiling_on_sc=False, needs_layout_passes=False),
        scratch_types=[
            pltpu.VMEM((CHUNK, H // NHP), jnp.float32),
            pltpu.VMEM((CHUNK, H // NHP), jnp.float32),
            pltpu.VMEM((S,), jnp.int32),
            pltpu.VMEM((S,), jnp.float32),
            pltpu.VMEM((G // (NW // B), H // NHP), jnp.float32),
            pltpu.SemaphoreType.DMA,
            pltpu.SemaphoreType.DMA,
        ],
    )
    out = run(feats_r, gids, vals)
    return out.reshape(B, G, H)


# speedup vs baseline: 2.1327x; 2.0750x over previous
import functools

import jax
import jax.numpy as jnp
from jax import lax
from jax.experimental import pallas as pl
from jax.experimental.pallas import tpu as pltpu
from jax.experimental.pallas import tpu_sc as plsc

G = 1024
CHUNK = 64
NHP = 2
NW = 32


def _lane_bcast(v16, i):
    return lax.gather(
        v16,
        jnp.full((16, 1), i, jnp.int32),
        lax.GatherDimensionNumbers(
            offset_dims=(),
            collapsed_slice_dims=(0,),
            start_index_map=(0,),
        ),
        slice_sizes=(1,),
        mode=lax.GatherScatterMode.PROMISE_IN_BOUNDS,
    )


def _sc_body(B, S, H, feats_hbm, gids_hbm, vals_hbm, out_hbm,
             buf0, buf1, gscan, vscan, acc, sem0, sem1):
    c = lax.axis_index("c")
    s = lax.axis_index("s")
    w = c * 16 + s

    wpb = NW // B
    gpw = G // wpb
    HP = H // NHP
    hv = HP // 16
    nch = S // CHUNK

    bw = w // wpb
    gr0 = (w % wpb) * gpw
    tok_base = bw * S

    pltpu.sync_copy(gids_hbm.at[pl.ds(tok_base, S)], gscan)
    pltpu.sync_copy(vals_hbm.at[pl.ds(tok_base, S)], vscan)
    lo_t = jnp.full((16,), gr0, jnp.int32)
    hi_t = jnp.full((16,), gr0 + gpw, jnp.int32)
    one = jnp.ones((16,), jnp.int32)
    zero = jnp.zeros((16,), jnp.int32)

    def count_step(i, carry):
        lo, hi = carry
        v = gscan[pl.ds(i * 16, 16)]
        return (lo + jnp.where(v < lo_t, one, zero),
                hi + jnp.where(v < hi_t, one, zero))

    lo_cnt, hi_cnt = lax.fori_loop(0, S // 16, count_step, (zero, zero))
    s_lo = jnp.int32(0)
    s_hi = jnp.int32(0)
    for j in range(16):
        s_lo = s_lo + lo_cnt[j]
        s_hi = s_hi + hi_cnt[j]

    p_lo = s_lo // (2 * CHUNK)
    p_hi = (s_hi + 2 * CHUNK - 1) // (2 * CHUNK)
    last_ch = nch - 1
    zvec = jnp.zeros((16,), jnp.float32)
    lo16 = jnp.full((16,), s_lo, jnp.int32)
    hi16 = jnp.full((16,), s_hi, jnp.int32)
    gr0_16 = jnp.full((16,), gr0, jnp.int32)

    for hp in range(NHP):
        col0 = hp * HP

        def zero_row(t, _):
            for k in range(hv):
                acc[t, pl.ds(k * 16, 16)] = zvec
            return 0

        lax.fori_loop(0, gpw, zero_row, 0)

        def issue(ch, buf, sem):
            chc = jnp.minimum(ch, last_ch)
            pltpu.async_copy(
                feats_hbm.at[pl.ds(tok_base + chc * CHUNK, CHUNK),
                             pl.ds(col0, HP)],
                buf, sem)

        def wait(buf, sem):
            pltpu.make_async_copy(
                feats_hbm.at[pl.ds(0, CHUNK), pl.ds(col0, HP)],
                buf, sem).wait()

        lane = lax.iota(jnp.int32, 16)

        def compute(ch, buf):
            def block_step(jj, _):
                off = ch * CHUNK + jj * 16
                g16 = gscan[pl.ds(off, 16)]
                v16 = vscan[pl.ds(off, 16)]
                tok16 = off + lane
                m = (tok16 >= lo16) & (tok16 < hi16)
                val16 = jnp.where(m, v16, 0.0)
                r16 = jnp.clip(g16 - gr0_16, 0, gpw - 1)
                valvs = [_lane_bcast(val16, i) for i in range(16)]
                rvecs = [_lane_bcast(r16, i) for i in range(16)]
                t0 = jj * 16

                @plsc.parallel_loop(0, hv, 1, unroll=2)
                def k_step(k):
                    col = k * 16 + lane
                    for i in range(16):
                        plsc.addupdate_scatter(
                            acc, [rvecs[i], col],
                            buf[t0 + i, pl.ds(k * 16, 16)] * valvs[i])

                return 0

            lax.fori_loop(0, CHUNK // 16, block_step, 0)

        issue(p_lo * 2, buf0, sem0)
        issue(p_lo * 2 + 1, buf1, sem1)

        def pair_step(p, _):
            ch0 = p * 2
            wait(buf0, sem0)
            compute(ch0, buf0)
            issue(ch0 + 2, buf0, sem0)
            wait(buf1, sem1)
            compute(ch0 + 1, buf1)
            issue(ch0 + 3, buf1, sem1)
            return 0

        lax.fori_loop(p_lo, p_hi, pair_step, 0)
        wait(buf0, sem0)
        wait(buf1, sem1)

        pltpu.sync_copy(
            acc, out_hbm.at[pl.ds(bw * G + gr0, gpw), pl.ds(col0, HP)])


def kernel(feats, indices, values):
    B, S, H = feats.shape
    feats_r = feats.reshape(B * S, H)
    gids = indices[1].astype(jnp.int32)
    vals = values.astype(jnp.float32)

    mesh = plsc.VectorSubcoreMesh(core_axis_name="c", subcore_axis_name="s")
    run = pl.kernel(
        functools.partial(_sc_body, B, S, H),
        out_type=jax.ShapeDtypeStruct((B * G, H), jnp.float32),
        mesh=mesh,
        compiler_params=pltpu.CompilerParams(
            use_tc_tiling_on_sc=False, needs_layout_passes=False),
        scratch_types=[
            pltpu.VMEM((CHUNK, H // NHP), jnp.float32),
            pltpu.VMEM((CHUNK, H // NHP), jnp.float32),
            pltpu.VMEM((S,), jnp.int32),
            pltpu.VMEM((S,), jnp.float32),
            pltpu.VMEM((G // (NW // B), H // NHP), jnp.float32),
            pltpu.SemaphoreType.DMA,
            pltpu.SemaphoreType.DMA,
        ],
    )
    out = run(feats_r, gids, vals)
    return out.reshape(B, G, H)


# speedup vs baseline: 2.1572x; 1.0115x over previous
import functools

import jax
import jax.numpy as jnp
from jax import lax
from jax.experimental import pallas as pl
from jax.experimental.pallas import tpu as pltpu
from jax.experimental.pallas import tpu_sc as plsc

G = 1024
CHUNK = 64
NHP = 2
NW = 32


def _lane_bcast(v16, i):
    return lax.gather(
        v16,
        jnp.full((16, 1), i, jnp.int32),
        lax.GatherDimensionNumbers(
            offset_dims=(),
            collapsed_slice_dims=(0,),
            start_index_map=(0,),
        ),
        slice_sizes=(1,),
        mode=lax.GatherScatterMode.PROMISE_IN_BOUNDS,
    )


def _sc_body(B, S, H, feats_hbm, gids_hbm, vals_hbm, out_hbm,
             buf0, buf1, gscan, vscan, acc, sem0, sem1):
    c = lax.axis_index("c")
    s = lax.axis_index("s")
    w = c * 16 + s

    wpb = NW // B
    gpw = G // wpb
    HP = H // NHP
    hv = HP // 16
    nch = S // CHUNK

    bw = w // wpb
    gr0 = (w % wpb) * gpw
    tok_base = bw * S

    pltpu.sync_copy(gids_hbm.at[pl.ds(tok_base, S)], gscan)
    pltpu.sync_copy(vals_hbm.at[pl.ds(tok_base, S)], vscan)
    lo_t = jnp.full((16,), gr0, jnp.int32)
    hi_t = jnp.full((16,), gr0 + gpw, jnp.int32)
    one = jnp.ones((16,), jnp.int32)
    zero = jnp.zeros((16,), jnp.int32)

    def count_step(i, carry):
        lo, hi = carry
        v = gscan[pl.ds(i * 16, 16)]
        return (lo + jnp.where(v < lo_t, one, zero),
                hi + jnp.where(v < hi_t, one, zero))

    lo_cnt, hi_cnt = lax.fori_loop(0, S // 16, count_step, (zero, zero))
    s_lo = jnp.int32(0)
    s_hi = jnp.int32(0)
    for j in range(16):
        s_lo = s_lo + lo_cnt[j]
        s_hi = s_hi + hi_cnt[j]

    p_lo = s_lo // (2 * CHUNK)
    p_hi = (s_hi + 2 * CHUNK - 1) // (2 * CHUNK)
    last_ch = nch - 1
    zvec = jnp.zeros((16,), jnp.float32)
    lo16 = jnp.full((16,), s_lo, jnp.int32)
    hi16 = jnp.full((16,), s_hi, jnp.int32)
    gr0_16 = jnp.full((16,), gr0, jnp.int32)

    for hp in range(NHP):
        col0 = hp * HP

        def zero_row(t, _):
            for k in range(hv):
                acc[t, pl.ds(k * 16, 16)] = zvec
            return 0

        lax.fori_loop(0, gpw, zero_row, 0)

        def issue(ch, buf, sem):
            chc = jnp.minimum(ch, last_ch)
            pltpu.async_copy(
                feats_hbm.at[pl.ds(tok_base + chc * CHUNK, CHUNK),
                             pl.ds(col0, HP)],
                buf, sem)

        def wait(buf, sem):
            pltpu.make_async_copy(
                feats_hbm.at[pl.ds(0, CHUNK), pl.ds(col0, HP)],
                buf, sem).wait()

        lane = lax.iota(jnp.int32, 16)

        def compute(ch, buf):
            def block_step(jj, _):
                off = ch * CHUNK + jj * 16
                g16 = gscan[pl.ds(off, 16)]
                v16 = vscan[pl.ds(off, 16)]
                tok16 = off + lane
                m = (tok16 >= lo16) & (tok16 < hi16)
                val16 = jnp.where(m, v16, 0.0)
                r16 = jnp.clip(g16 - gr0_16, 0, gpw - 1)
                valvs = [_lane_bcast(val16, i) for i in range(16)]
                rvecs = [_lane_bcast(r16, i) for i in range(16)]
                t0 = jj * 16

                @plsc.parallel_loop(0, hv, 1, unroll=4)
                def k_step(k):
                    col = k * 16 + lane
                    for i in range(16):
                        plsc.addupdate_scatter(
                            acc, [rvecs[i], col],
                            buf[t0 + i, pl.ds(k * 16, 16)] * valvs[i])

                return 0

            lax.fori_loop(0, CHUNK // 16, block_step, 0)

        issue(p_lo * 2, buf0, sem0)
        issue(p_lo * 2 + 1, buf1, sem1)

        def pair_step(p, _):
            ch0 = p * 2
            wait(buf0, sem0)
            compute(ch0, buf0)
            issue(ch0 + 2, buf0, sem0)
            wait(buf1, sem1)
            compute(ch0 + 1, buf1)
            issue(ch0 + 3, buf1, sem1)
            return 0

        lax.fori_loop(p_lo, p_hi, pair_step, 0)
        wait(buf0, sem0)
        wait(buf1, sem1)

        pltpu.sync_copy(
            acc, out_hbm.at[pl.ds(bw * G + gr0, gpw), pl.ds(col0, HP)])


def kernel(feats, indices, values):
    B, S, H = feats.shape
    feats_r = feats.reshape(B * S, H)
    gids = indices[1].astype(jnp.int32)
    vals = values.astype(jnp.float32)

    mesh = plsc.VectorSubcoreMesh(core_axis_name="c", subcore_axis_name="s")
    run = pl.kernel(
        functools.partial(_sc_body, B, S, H),
        out_type=jax.ShapeDtypeStruct((B * G, H), jnp.float32),
        mesh=mesh,
        compiler_params=pltpu.CompilerParams(
            use_tc_tiling_on_sc=False, needs_layout_passes=False),
        scratch_types=[
            pltpu.VMEM((CHUNK, H // NHP), jnp.float32),
            pltpu.VMEM((CHUNK, H // NHP), jnp.float32),
            pltpu.VMEM((S,), jnp.int32),
            pltpu.VMEM((S,), jnp.float32),
            pltpu.VMEM((G // (NW // B), H // NHP), jnp.float32),
            pltpu.SemaphoreType.DMA,
            pltpu.SemaphoreType.DMA,
        ],
    )
    out = run(feats_r, gids, vals)
    return out.reshape(B, G, H)
